# Initial kernel scaffold; baseline (speedup 1.0000x reference)
import jax
import jax.numpy as jnp
from jax.experimental import pallas as pl


def kernel(x, q, Wk, Wv, Wm1, Wm2, Wexp, alpha):
    raise NotImplementedError("write your pallas kernel here")


# speedup vs baseline: 2.0234x; 2.0234x over previous
import jax
import jax.numpy as jnp
from jax.experimental import pallas as pl

K = 8
HEADS = 2
DH = 64
MLP = 64
C = 192
STEPS = 2
TOPN = 2
TEMP = 1.5
HW = 28 * 28

_BF = jnp.bfloat16
_F32 = jnp.float32


def _bdot(a_bf, b_bf):
    return jax.lax.dot_general(a_bf, b_bf, (((1,), (0,)), ((), ())),
                               preferred_element_type=_F32)


def _sample_kernel(x_ref, qmat_ref, wkv_ref, wm1_ref, wm2_ref, wexp_ref,
                   alpha_ref, out_ref):
    y = x_ref[0]
    ealpha = jnp.exp(alpha_ref[0, 0])
    eidx = jax.lax.broadcasted_iota(jnp.int32, (1, K), 1)
    allowed = jnp.ones((1, K), dtype=_F32)

    for _t in range(STEPS):
        y_bf = y.astype(_BF)
        kv = _bdot(y_bf, wkv_ref[:])
        k = kv[:, :HEADS * DH]
        v = kv[:, HEADS * DH:]

        al = _bdot(k.astype(_BF), qmat_ref[:]) / jnp.sqrt(jnp.float32(DH))
        al = al - jnp.max(al, axis=0, keepdims=True)
        ex = jnp.exp(al)
        attn = ex / jnp.sum(ex, axis=0, keepdims=True)

        attn_bf = attn.astype(_BF)
        v_bf = v.astype(_BF)
        pooled = jnp.concatenate(
            [jax.lax.dot_general(attn_bf[:, h:h + 1], v_bf[:, h * DH:(h + 1) * DH],
                                 (((0,), (0,)), ((), ())),
                                 preferred_element_type=_F32)
             for h in range(HEADS)], axis=1)

        hmid = _bdot(pooled.astype(_BF), wm1_ref[:])
        hmid = hmid * jax.nn.sigmoid(hmid)
        logits = _bdot(hmid.astype(_BF), wm2_ref[:])

        masked = jnp.where(allowed > 0.5, logits, jnp.float32(-1e9)) / TEMP
        masked = masked - jnp.max(masked)
        pe = jnp.exp(masked)
        probs = pe / jnp.sum(pe)

        v0 = jnp.max(probs)
        i0 = jnp.min(jnp.where(probs == v0, eidx, K))
        p2 = jnp.where(eidx == i0, jnp.float32(-1.0), probs)
        v1 = jnp.max(p2)
        i1 = jnp.min(jnp.where(p2 == v1, eidx, K))
        denom = (v0 + v1) + jnp.float32(1e-9)
        w0 = v0 / denom
        w1 = v1 / denom

        am = (attn[:, 0:1] + attn[:, 1:2]) / 2.0
        x_mod = y * (1.0 + ealpha * am)

        we0 = wexp_ref[pl.ds(i0, 1), :, :].reshape(C, C)
        we1 = wexp_ref[pl.ds(i1, 1), :, :].reshape(C, C)
        ypair = _bdot(x_mod.astype(_BF), jnp.concatenate([we0, we1], axis=1))
        y = w0 * ypair[:, :C] + w1 * ypair[:, C:]

        allowed = jnp.where((eidx == i0) & (i0 != K - 1), jnp.float32(0.0),
                            allowed)

    out_ref[0] = y


@jax.jit
def kernel(x, q, Wk, Wv, Wm1, Wm2, Wexp, alpha):
    B, H, W, Cc = x.shape
    xf = x.reshape(B, H * W, Cc)
    hd_ids = jnp.arange(HEADS * DH, dtype=jnp.int32) // DH
    qmat = jnp.where(hd_ids[:, None] == jnp.arange(HEADS, dtype=jnp.int32)[None, :],
                     q.reshape(HEADS * DH)[:, None], 0.0)
    wkv = jnp.concatenate([Wk, Wv], axis=1)
    alpha2 = alpha.reshape(1, 1)
    out = pl.pallas_call(
        _sample_kernel,
        grid=(B,),
        in_specs=[
            pl.BlockSpec((1, H * W, Cc), lambda b: (b, 0, 0)),
            pl.BlockSpec((HEADS * DH, HEADS), lambda b: (0, 0)),
            pl.BlockSpec((Cc, 2 * HEADS * DH), lambda b: (0, 0)),
            pl.BlockSpec((HEADS * DH, MLP), lambda b: (0, 0)),
            pl.BlockSpec((MLP, K), lambda b: (0, 0)),
            pl.BlockSpec((K, Cc, Cc), lambda b: (0, 0, 0)),
            pl.BlockSpec((1, 1), lambda b: (0, 0)),
        ],
        out_specs=pl.BlockSpec((1, H * W, Cc), lambda b: (b, 0, 0)),
        out_shape=jax.ShapeDtypeStruct((B, H * W, Cc), jnp.float32),
    )(xf, qmat.astype(_BF), wkv.astype(_BF), Wm1.astype(_BF), Wm2.astype(_BF),
      Wexp.astype(_BF), alpha2)
    return out.reshape(B, H, W, Cc)


# speedup vs baseline: 3.6525x; 1.8051x over previous
import jax
import jax.numpy as jnp
from jax.experimental import pallas as pl

K = 8
HEADS = 2
DH = 64
MLP = 64
C = 192
STEPS = 2
TOPN = 2
TEMP = 1.5
HW = 28 * 28
S = 4

_BF = jnp.bfloat16
_F32 = jnp.float32


def _bdot(a_bf, b_bf):
    return jax.lax.dot_general(a_bf, b_bf, (((1,), (0,)), ((), ())),
                               preferred_element_type=_F32)


def _block_kernel(x_ref, qmat_ref, wkv_ref, wm1_ref, wm2_ref, wexp_ref,
                  alpha_ref, out_ref):
    ealpha = jnp.exp(alpha_ref[0, 0])
    eidx = jax.lax.broadcasted_iota(jnp.int32, (S, K), 1)
    allowed = jnp.ones((S, K), dtype=_F32)
    y = x_ref[:].reshape(S * HW, C)

    for _t in range(STEPS):
        y_bf = y.astype(_BF)
        kv = _bdot(y_bf, wkv_ref[:])
        al_all = _bdot(kv[:, :HEADS * DH].astype(_BF), qmat_ref[:]) \
            / jnp.sqrt(jnp.float32(DH))
        v_all_bf = kv[:, HEADS * DH:].astype(_BF)

        attns = []
        pooled_rows = []
        for s in range(S):
            al = al_all[s * HW:(s + 1) * HW]
            al = al - jnp.max(al, axis=0, keepdims=True)
            ex = jnp.exp(al)
            attn = ex / jnp.sum(ex, axis=0, keepdims=True)
            attns.append(attn)
            attn_bf = attn.astype(_BF)
            v_bf = v_all_bf[s * HW:(s + 1) * HW]
            pooled = jnp.concatenate(
                [jax.lax.dot_general(attn_bf[:, h:h + 1],
                                     v_bf[:, h * DH:(h + 1) * DH],
                                     (((0,), (0,)), ((), ())),
                                     preferred_element_type=_F32)
                 for h in range(HEADS)], axis=1)
            pooled_rows.append(pooled)
        pooled_all = jnp.concatenate(pooled_rows, axis=0)

        hmid = _bdot(pooled_all.astype(_BF), wm1_ref[:])
        hmid = hmid * jax.nn.sigmoid(hmid)
        logits = _bdot(hmid.astype(_BF), wm2_ref[:])

        masked = jnp.where(allowed > 0.5, logits, jnp.float32(-1e9)) / TEMP
        masked = masked - jnp.max(masked, axis=1, keepdims=True)
        pe = jnp.exp(masked)
        probs = pe / jnp.sum(pe, axis=1, keepdims=True)

        v0 = jnp.max(probs, axis=1, keepdims=True)
        i0 = jnp.min(jnp.where(probs == v0, eidx, K), axis=1, keepdims=True)
        p2 = jnp.where(eidx == i0, jnp.float32(-1.0), probs)
        v1 = jnp.max(p2, axis=1, keepdims=True)
        i1 = jnp.min(jnp.where(p2 == v1, eidx, K), axis=1, keepdims=True)
        denom = (v0 + v1) + jnp.float32(1e-9)
        w0 = v0 / denom
        w1 = v1 / denom

        am_all = jnp.concatenate(
            [(a[:, 0:1] + a[:, 1:2]) / 2.0 for a in attns], axis=0)
        x_mod_bf = (y * (1.0 + ealpha * am_all)).astype(_BF)

        outs = []
        for s in range(S):
            we0 = wexp_ref[pl.ds(i0[s, 0], 1), :, :].reshape(C, C)
            we1 = wexp_ref[pl.ds(i1[s, 0], 1), :, :].reshape(C, C)
            ypair = _bdot(x_mod_bf[s * HW:(s + 1) * HW],
                          jnp.concatenate([we0, we1], axis=1))
            outs.append(w0[s, 0] * ypair[:, :C] + w1[s, 0] * ypair[:, C:])
        y = jnp.concatenate(outs, axis=0)

        allowed = jnp.where((eidx == i0) & (i0 != K - 1), jnp.float32(0.0),
                            allowed)

    out_ref[:] = y.reshape(S, HW, C)


@jax.jit
def kernel(x, q, Wk, Wv, Wm1, Wm2, Wexp, alpha):
    B, H, W, Cc = x.shape
    xf = x.reshape(B, H * W, Cc)
    hd_ids = jnp.arange(HEADS * DH, dtype=jnp.int32) // DH
    qmat = jnp.where(hd_ids[:, None] == jnp.arange(HEADS, dtype=jnp.int32)[None, :],
                     q.reshape(HEADS * DH)[:, None], 0.0)
    wkv = jnp.concatenate([Wk, Wv], axis=1)
    alpha2 = alpha.reshape(1, 1)
    out = pl.pallas_call(
        _block_kernel,
        grid=(B // S,),
        in_specs=[
            pl.BlockSpec((S, H * W, Cc), lambda b: (b, 0, 0)),
            pl.BlockSpec((HEADS * DH, HEADS), lambda b: (0, 0)),
            pl.BlockSpec((Cc, 2 * HEADS * DH), lambda b: (0, 0)),
            pl.BlockSpec((HEADS * DH, MLP), lambda b: (0, 0)),
            pl.BlockSpec((MLP, K), lambda b: (0, 0)),
            pl.BlockSpec((K, Cc, Cc), lambda b: (0, 0, 0)),
            pl.BlockSpec((1, 1), lambda b: (0, 0)),
        ],
        out_specs=pl.BlockSpec((S, H * W, Cc), lambda b: (b, 0, 0)),
        out_shape=jax.ShapeDtypeStruct((B, H * W, Cc), jnp.float32),
    )(xf, qmat.astype(_BF), wkv.astype(_BF), Wm1.astype(_BF), Wm2.astype(_BF),
      Wexp.astype(_BF), alpha2)
    return out.reshape(B, H, W, Cc)


# speedup vs baseline: 3.8548x; 1.0554x over previous
import jax
import jax.numpy as jnp
from jax.experimental import pallas as pl

K = 8
HEADS = 2
DH = 64
MLP = 64
C = 192
STEPS = 2
TOPN = 2
TEMP = 1.5
HW = 28 * 28
S = 8

_BF = jnp.bfloat16
_F32 = jnp.float32


def _bdot(a_bf, b_bf):
    return jax.lax.dot_general(a_bf, b_bf, (((1,), (0,)), ((), ())),
                               preferred_element_type=_F32)


def _block_kernel(x_ref, qmat_ref, wkv_ref, wm1_ref, wm2_ref, wexp_ref,
                  alpha_ref, out_ref):
    ealpha = jnp.exp(alpha_ref[0, 0])
    eidx = jax.lax.broadcasted_iota(jnp.int32, (S, K), 1)
    allowed = jnp.ones((S, K), dtype=_F32)
    y = x_ref[:].reshape(S * HW, C)

    for _t in range(STEPS):
        y_bf = y.astype(_BF)
        kv = _bdot(y_bf, wkv_ref[:])
        al_all = _bdot(kv[:, :HEADS * DH].astype(_BF), qmat_ref[:]) \
            / jnp.sqrt(jnp.float32(DH))
        v_all_bf = kv[:, HEADS * DH:].astype(_BF)

        attns = []
        pooled_rows = []
        for s in range(S):
            al = al_all[s * HW:(s + 1) * HW]
            al = al - jnp.max(al, axis=0, keepdims=True)
            ex = jnp.exp(al)
            attn = ex / jnp.sum(ex, axis=0, keepdims=True)
            attns.append(attn)
            attn_bf = attn.astype(_BF)
            v_bf = v_all_bf[s * HW:(s + 1) * HW]
            pooled = jnp.concatenate(
                [jax.lax.dot_general(attn_bf[:, h:h + 1],
                                     v_bf[:, h * DH:(h + 1) * DH],
                                     (((0,), (0,)), ((), ())),
                                     preferred_element_type=_F32)
                 for h in range(HEADS)], axis=1)
            pooled_rows.append(pooled)
        pooled_all = jnp.concatenate(pooled_rows, axis=0)

        hmid = _bdot(pooled_all.astype(_BF), wm1_ref[:])
        hmid = hmid * jax.nn.sigmoid(hmid)
        logits = _bdot(hmid.astype(_BF), wm2_ref[:])

        masked = jnp.where(allowed > 0.5, logits, jnp.float32(-1e9)) / TEMP
        masked = masked - jnp.max(masked, axis=1, keepdims=True)
        pe = jnp.exp(masked)
        probs = pe / jnp.sum(pe, axis=1, keepdims=True)

        v0 = jnp.max(probs, axis=1, keepdims=True)
        i0 = jnp.min(jnp.where(probs == v0, eidx, K), axis=1, keepdims=True)
        p2 = jnp.where(eidx == i0, jnp.float32(-1.0), probs)
        v1 = jnp.max(p2, axis=1, keepdims=True)
        i1 = jnp.min(jnp.where(p2 == v1, eidx, K), axis=1, keepdims=True)
        denom = (v0 + v1) + jnp.float32(1e-9)
        w0 = v0 / denom
        w1 = v1 / denom

        am_all = jnp.concatenate(
            [(a[:, 0:1] + a[:, 1:2]) / 2.0 for a in attns], axis=0)
        x_mod_bf = (y * (1.0 + ealpha * am_all)).astype(_BF)

        outs = []
        for s in range(S):
            we0 = wexp_ref[pl.ds(i0[s, 0], 1), :, :].reshape(C, C)
            we1 = wexp_ref[pl.ds(i1[s, 0], 1), :, :].reshape(C, C)
            ypair = _bdot(x_mod_bf[s * HW:(s + 1) * HW],
                          jnp.concatenate([we0, we1], axis=1))
            outs.append(w0[s, 0] * ypair[:, :C] + w1[s, 0] * ypair[:, C:])
        y = jnp.concatenate(outs, axis=0)

        allowed = jnp.where((eidx == i0) & (i0 != K - 1), jnp.float32(0.0),
                            allowed)

    out_ref[:] = y.reshape(S, HW, C)


@jax.jit
def kernel(x, q, Wk, Wv, Wm1, Wm2, Wexp, alpha):
    B, H, W, Cc = x.shape
    xf = x.reshape(B, H * W, Cc)
    hd_ids = jnp.arange(HEADS * DH, dtype=jnp.int32) // DH
    qmat = jnp.where(hd_ids[:, None] == jnp.arange(HEADS, dtype=jnp.int32)[None, :],
                     q.reshape(HEADS * DH)[:, None], 0.0)
    wkv = jnp.concatenate([Wk, Wv], axis=1)
    alpha2 = alpha.reshape(1, 1)
    out = pl.pallas_call(
        _block_kernel,
        grid=(B // S,),
        in_specs=[
            pl.BlockSpec((S, H * W, Cc), lambda b: (b, 0, 0)),
            pl.BlockSpec((HEADS * DH, HEADS), lambda b: (0, 0)),
            pl.BlockSpec((Cc, 2 * HEADS * DH), lambda b: (0, 0)),
            pl.BlockSpec((HEADS * DH, MLP), lambda b: (0, 0)),
            pl.BlockSpec((MLP, K), lambda b: (0, 0)),
            pl.BlockSpec((K, Cc, Cc), lambda b: (0, 0, 0)),
            pl.BlockSpec((1, 1), lambda b: (0, 0)),
        ],
        out_specs=pl.BlockSpec((S, H * W, Cc), lambda b: (b, 0, 0)),
        out_shape=jax.ShapeDtypeStruct((B, H * W, Cc), jnp.float32),
    )(xf, qmat.astype(_BF), wkv.astype(_BF), Wm1.astype(_BF), Wm2.astype(_BF),
      Wexp.astype(_BF), alpha2)
    return out.reshape(B, H, W, Cc)


# speedup vs baseline: 3.8654x; 1.0027x over previous
import jax
import jax.numpy as jnp
from jax.experimental import pallas as pl
from jax.experimental.pallas import tpu as pltpu

K = 8
HEADS = 2
DH = 64
MLP = 64
C = 192
STEPS = 2
TOPN = 2
TEMP = 1.5
HW = 28 * 28
S = 8

_BF = jnp.bfloat16
_F32 = jnp.float32


def _bdot(a_bf, b_bf):
    return jax.lax.dot_general(a_bf, b_bf, (((1,), (0,)), ((), ())),
                               preferred_element_type=_F32)


def _block_kernel(x_ref, qmat_ref, wkv_ref, wm1_ref, wm2_ref, wexp_ref,
                  alpha_ref, out_ref, y_ref):
    ealpha = jnp.exp(alpha_ref[0, 0])
    eidx = jax.lax.broadcasted_iota(jnp.int32, (S, K), 1)
    allowed = jnp.ones((S, K), dtype=_F32)

    for _t in range(STEPS):
        y = x_ref[:].reshape(S * HW, C) if _t == 0 else y_ref[:]
        y_bf = y.astype(_BF)
        kv = _bdot(y_bf, wkv_ref[:])
        al_all = _bdot(kv[:, :HEADS * DH].astype(_BF), qmat_ref[:]) \
            / jnp.sqrt(jnp.float32(DH))
        v_all_bf = kv[:, HEADS * DH:].astype(_BF)

        attns = []
        pooled_rows = []
        for s in range(S):
            al = al_all[s * HW:(s + 1) * HW]
            al = al - jnp.max(al, axis=0, keepdims=True)
            ex = jnp.exp(al)
            attn = ex / jnp.sum(ex, axis=0, keepdims=True)
            attns.append(attn)
            attn_bf = attn.astype(_BF)
            v_bf = v_all_bf[s * HW:(s + 1) * HW]
            pooled = jnp.concatenate(
                [jax.lax.dot_general(attn_bf[:, h:h + 1],
                                     v_bf[:, h * DH:(h + 1) * DH],
                                     (((0,), (0,)), ((), ())),
                                     preferred_element_type=_F32)
                 for h in range(HEADS)], axis=1)
            pooled_rows.append(pooled)
        pooled_all = jnp.concatenate(pooled_rows, axis=0)

        hmid = _bdot(pooled_all.astype(_BF), wm1_ref[:])
        hmid = hmid * jax.nn.sigmoid(hmid)
        logits = _bdot(hmid.astype(_BF), wm2_ref[:])

        masked = jnp.where(allowed > 0.5, logits, jnp.float32(-1e9)) / TEMP
        masked = masked - jnp.max(masked, axis=1, keepdims=True)
        pe = jnp.exp(masked)
        probs = pe / jnp.sum(pe, axis=1, keepdims=True)

        v0 = jnp.max(probs, axis=1, keepdims=True)
        i0 = jnp.min(jnp.where(probs == v0, eidx, K), axis=1, keepdims=True)
        p2 = jnp.where(eidx == i0, jnp.float32(-1.0), probs)
        v1 = jnp.max(p2, axis=1, keepdims=True)
        i1 = jnp.min(jnp.where(p2 == v1, eidx, K), axis=1, keepdims=True)
        denom = (v0 + v1) + jnp.float32(1e-9)
        w0 = v0 / denom
        w1 = v1 / denom

        for s in range(S):
            a = attns[s]
            am = (a[:, 0:1] + a[:, 1:2]) / 2.0
            x_mod_bf = (y[s * HW:(s + 1) * HW] * (1.0 + ealpha * am)).astype(_BF)
            we0 = wexp_ref[pl.ds(i0[s, 0], 1), :, :].reshape(C, C)
            we1 = wexp_ref[pl.ds(i1[s, 0], 1), :, :].reshape(C, C)
            ypair = _bdot(x_mod_bf, jnp.concatenate([we0, we1], axis=1))
            out_s = w0[s, 0] * ypair[:, :C] + w1[s, 0] * ypair[:, C:]
            if _t == STEPS - 1:
                out_ref[s] = out_s
            else:
                y_ref[s * HW:(s + 1) * HW, :] = out_s

        allowed = jnp.where((eidx == i0) & (i0 != K - 1), jnp.float32(0.0),
                            allowed)


@jax.jit
def kernel(x, q, Wk, Wv, Wm1, Wm2, Wexp, alpha):
    B, H, W, Cc = x.shape
    xf = x.reshape(B, H * W, Cc)
    hd_ids = jnp.arange(HEADS * DH, dtype=jnp.int32) // DH
    qmat = jnp.where(hd_ids[:, None] == jnp.arange(HEADS, dtype=jnp.int32)[None, :],
                     q.reshape(HEADS * DH)[:, None], 0.0)
    wkv = jnp.concatenate([Wk, Wv], axis=1)
    alpha2 = alpha.reshape(1, 1)
    out = pl.pallas_call(
        _block_kernel,
        grid=(B // S,),
        in_specs=[
            pl.BlockSpec((S, H * W, Cc), lambda b: (b, 0, 0)),
            pl.BlockSpec((HEADS * DH, HEADS), lambda b: (0, 0)),
            pl.BlockSpec((Cc, 2 * HEADS * DH), lambda b: (0, 0)),
            pl.BlockSpec((HEADS * DH, MLP), lambda b: (0, 0)),
            pl.BlockSpec((MLP, K), lambda b: (0, 0)),
            pl.BlockSpec((K, Cc, Cc), lambda b: (0, 0, 0)),
            pl.BlockSpec((1, 1), lambda b: (0, 0)),
        ],
        out_specs=pl.BlockSpec((S, H * W, Cc), lambda b: (b, 0, 0)),
        out_shape=jax.ShapeDtypeStruct((B, H * W, Cc), jnp.float32),
        scratch_shapes=[pltpu.VMEM((S * HW, Cc), jnp.float32)],
    )(xf, qmat.astype(_BF), wkv.astype(_BF), Wm1.astype(_BF), Wm2.astype(_BF),
      Wexp.astype(_BF), alpha2)
    return out.reshape(B, H, W, Cc)


# speedup vs baseline: 3.9459x; 1.0208x over previous
import jax
import jax.numpy as jnp
from jax.experimental import pallas as pl
from jax.experimental.pallas import tpu as pltpu

K = 8
HEADS = 2
DH = 64
MLP = 64
C = 192
STEPS = 2
TOPN = 2
TEMP = 1.5
HW = 28 * 28
S = 8

_BF = jnp.bfloat16
_F32 = jnp.float32


def _bdot(a_bf, b_bf):
    return jax.lax.dot_general(a_bf, b_bf, (((1,), (0,)), ((), ())),
                               preferred_element_type=_F32)


def _block_kernel(x_ref, qmat_ref, wkv_ref, wm1_ref, wm2_ref, wexp_ref,
                  alpha_ref, out_ref, y_ref):
    ealpha = jnp.exp(alpha_ref[0, 0])
    eidx = jax.lax.broadcasted_iota(jnp.int32, (S, K), 1)
    allowed = jnp.ones((S, K), dtype=_F32)

    for _t in range(STEPS):
        y = x_ref[:].reshape(S * HW, C) if _t == 0 else y_ref[:]
        y_bf = y.astype(_BF)
        kv = _bdot(y_bf, wkv_ref[:])
        al_all = _bdot(kv[:, :HEADS * DH].astype(_BF), qmat_ref[:]) \
            / jnp.sqrt(jnp.float32(DH))
        v_all_bf = kv[:, HEADS * DH:].astype(_BF)

        attns = []
        pooled_rows = []
        for s in range(S):
            al = al_all[s * HW:(s + 1) * HW]
            al = al - jnp.max(al, axis=0, keepdims=True)
            ex = jnp.exp(al)
            attn = ex / jnp.sum(ex, axis=0, keepdims=True)
            attns.append(attn)
            attn_bf = attn.astype(_BF)
            v_bf = v_all_bf[s * HW:(s + 1) * HW]
            pooled = jnp.concatenate(
                [jax.lax.dot_general(attn_bf[:, h:h + 1],
                                     v_bf[:, h * DH:(h + 1) * DH],
                                     (((0,), (0,)), ((), ())),
                                     preferred_element_type=_F32)
                 for h in range(HEADS)], axis=1)
            pooled_rows.append(pooled)
        pooled_all = jnp.concatenate(pooled_rows, axis=0)

        hmid = _bdot(pooled_all.astype(_BF), wm1_ref[:])
        hmid = hmid * jax.nn.sigmoid(hmid)
        logits = _bdot(hmid.astype(_BF), wm2_ref[:])

        masked = jnp.where(allowed > 0.5, logits, jnp.float32(-1e9)) / TEMP
        masked = masked - jnp.max(masked, axis=1, keepdims=True)
        pe = jnp.exp(masked)
        probs = pe / jnp.sum(pe, axis=1, keepdims=True)

        v0 = jnp.max(probs, axis=1, keepdims=True)
        i0 = jnp.min(jnp.where(probs == v0, eidx, K), axis=1, keepdims=True)
        p2 = jnp.where(eidx == i0, jnp.float32(-1.0), probs)
        v1 = jnp.max(p2, axis=1, keepdims=True)
        i1 = jnp.min(jnp.where(p2 == v1, eidx, K), axis=1, keepdims=True)
        denom = (v0 + v1) + jnp.float32(1e-9)
        w0 = v0 / denom
        w1 = v1 / denom

        for s in range(S):
            a = attns[s]
            am = (a[:, 0:1] + a[:, 1:2]) / 2.0
            x_mod_bf = (y[s * HW:(s + 1) * HW] * (1.0 + ealpha * am)).astype(_BF)
            we0 = wexp_ref[pl.ds(i0[s, 0], 1), :, :].reshape(C, C)
            we1 = wexp_ref[pl.ds(i1[s, 0], 1), :, :].reshape(C, C)
            ypair = _bdot(x_mod_bf, jnp.concatenate([we0, we1], axis=1))
            out_s = w0[s, 0] * ypair[:, :C] + w1[s, 0] * ypair[:, C:]
            if _t == STEPS - 1:
                out_ref[s] = out_s.reshape(28, 28, C)
            else:
                y_ref[s * HW:(s + 1) * HW, :] = out_s

        allowed = jnp.where((eidx == i0) & (i0 != K - 1), jnp.float32(0.0),
                            allowed)


@jax.jit
def kernel(x, q, Wk, Wv, Wm1, Wm2, Wexp, alpha):
    B, H, W, Cc = x.shape
    hd_ids = jnp.arange(HEADS * DH, dtype=jnp.int32) // DH
    qmat = jnp.where(hd_ids[:, None] == jnp.arange(HEADS, dtype=jnp.int32)[None, :],
                     q.reshape(HEADS * DH)[:, None], 0.0)
    wkv = jnp.concatenate([Wk, Wv], axis=1)
    alpha2 = alpha.reshape(1, 1)
    out = pl.pallas_call(
        _block_kernel,
        grid=(B // S,),
        in_specs=[
            pl.BlockSpec((S, H, W, Cc), lambda b: (b, 0, 0, 0)),
            pl.BlockSpec((HEADS * DH, HEADS), lambda b: (0, 0)),
            pl.BlockSpec((Cc, 2 * HEADS * DH), lambda b: (0, 0)),
            pl.BlockSpec((HEADS * DH, MLP), lambda b: (0, 0)),
            pl.BlockSpec((MLP, K), lambda b: (0, 0)),
            pl.BlockSpec((K, Cc, Cc), lambda b: (0, 0, 0)),
            pl.BlockSpec((1, 1), lambda b: (0, 0)),
        ],
        out_specs=pl.BlockSpec((S, H, W, Cc), lambda b: (b, 0, 0, 0)),
        out_shape=jax.ShapeDtypeStruct((B, H, W, Cc), jnp.float32),
        scratch_shapes=[pltpu.VMEM((S * HW, Cc), jnp.float32)],
    )(x, qmat.astype(_BF), wkv.astype(_BF), Wm1.astype(_BF), Wm2.astype(_BF),
      Wexp.astype(_BF), alpha2)
    return out


# speedup vs baseline: 6.2794x; 1.5914x over previous
import jax
import jax.numpy as jnp
from jax.experimental import pallas as pl
from jax.experimental.pallas import tpu as pltpu

K = 8
HEADS = 2
DH = 64
MLP = 64
C = 192
STEPS = 2
TOPN = 2
TEMP = 1.5
HW = 28 * 28
S = 8

_BF = jnp.bfloat16
_F32 = jnp.float32


def _bdot(a_bf, b_bf):
    return jax.lax.dot_general(a_bf, b_bf, (((1,), (0,)), ((), ())),
                               preferred_element_type=_F32)


def _block_kernel(x_ref, qmat_ref, wkv_ref, wm1_ref, wm2_ref, wexp_ref,
                  alpha_ref, out_ref, y_ref):
    ealpha = jnp.exp(alpha_ref[0, 0])
    eidx = jax.lax.broadcasted_iota(jnp.int32, (S, K), 1)
    allowed = jnp.ones((S, K), dtype=_F32)

    x_tm = x_ref[:].reshape(HW, S, C)
    x_sm = jnp.swapaxes(x_tm, 0, 1).reshape(S * HW, C)

    for _t in range(STEPS):
        y = x_sm if _t == 0 else y_ref[:]
        y_bf = y.astype(_BF)
        kv = _bdot(y_bf, wkv_ref[:])
        al_all = _bdot(kv[:, :HEADS * DH].astype(_BF), qmat_ref[:]) \
            / jnp.sqrt(jnp.float32(DH))
        v_all_bf = kv[:, HEADS * DH:].astype(_BF)

        attns = []
        pooled_rows = []
        for s in range(S):
            al = al_all[s * HW:(s + 1) * HW]
            al = al - jnp.max(al, axis=0, keepdims=True)
            ex = jnp.exp(al)
            attn = ex / jnp.sum(ex, axis=0, keepdims=True)
            attns.append(attn)
            attn_bf = attn.astype(_BF)
            v_bf = v_all_bf[s * HW:(s + 1) * HW]
            pooled = jnp.concatenate(
                [jax.lax.dot_general(attn_bf[:, h:h + 1],
                                     v_bf[:, h * DH:(h + 1) * DH],
                                     (((0,), (0,)), ((), ())),
                                     preferred_element_type=_F32)
                 for h in range(HEADS)], axis=1)
            pooled_rows.append(pooled)
        pooled_all = jnp.concatenate(pooled_rows, axis=0)

        hmid = _bdot(pooled_all.astype(_BF), wm1_ref[:])
        hmid = hmid * jax.nn.sigmoid(hmid)
        logits = _bdot(hmid.astype(_BF), wm2_ref[:])

        masked = jnp.where(allowed > 0.5, logits, jnp.float32(-1e9)) / TEMP
        masked = masked - jnp.max(masked, axis=1, keepdims=True)
        pe = jnp.exp(masked)
        probs = pe / jnp.sum(pe, axis=1, keepdims=True)

        v0 = jnp.max(probs, axis=1, keepdims=True)
        i0 = jnp.min(jnp.where(probs == v0, eidx, K), axis=1, keepdims=True)
        p2 = jnp.where(eidx == i0, jnp.float32(-1.0), probs)
        v1 = jnp.max(p2, axis=1, keepdims=True)
        i1 = jnp.min(jnp.where(p2 == v1, eidx, K), axis=1, keepdims=True)
        denom = (v0 + v1) + jnp.float32(1e-9)
        w0 = v0 / denom
        w1 = v1 / denom

        for s in range(S):
            a = attns[s]
            am = (a[:, 0:1] + a[:, 1:2]) / 2.0
            x_mod_bf = (y[s * HW:(s + 1) * HW] * (1.0 + ealpha * am)).astype(_BF)
            we0 = wexp_ref[pl.ds(i0[s, 0], 1), :, :].reshape(C, C)
            we1 = wexp_ref[pl.ds(i1[s, 0], 1), :, :].reshape(C, C)
            ypair = _bdot(x_mod_bf, jnp.concatenate([we0, we1], axis=1))
            out_s = w0[s, 0] * ypair[:, :C] + w1[s, 0] * ypair[:, C:]
            y_ref[s * HW:(s + 1) * HW, :] = out_s

        allowed = jnp.where((eidx == i0) & (i0 != K - 1), jnp.float32(0.0),
                            allowed)

    y_fin = y_ref[:].reshape(S, HW, C)
    out_ref[:] = jnp.swapaxes(y_fin, 0, 1).reshape(28, 28, S, C)


@jax.jit
def kernel(x, q, Wk, Wv, Wm1, Wm2, Wexp, alpha):
    B, H, W, Cc = x.shape
    hd_ids = jnp.arange(HEADS * DH, dtype=jnp.int32) // DH
    qmat = jnp.where(hd_ids[:, None] == jnp.arange(HEADS, dtype=jnp.int32)[None, :],
                     q.reshape(HEADS * DH)[:, None], 0.0)
    wkv = jnp.concatenate([Wk, Wv], axis=1)
    alpha2 = alpha.reshape(1, 1)
    xt = jnp.transpose(x, (1, 2, 0, 3))
    out = pl.pallas_call(
        _block_kernel,
        grid=(B // S,),
        in_specs=[
            pl.BlockSpec((H, W, S, Cc), lambda b: (0, 0, b, 0)),
            pl.BlockSpec((HEADS * DH, HEADS), lambda b: (0, 0)),
            pl.BlockSpec((Cc, 2 * HEADS * DH), lambda b: (0, 0)),
            pl.BlockSpec((HEADS * DH, MLP), lambda b: (0, 0)),
            pl.BlockSpec((MLP, K), lambda b: (0, 0)),
            pl.BlockSpec((K, Cc, Cc), lambda b: (0, 0, 0)),
            pl.BlockSpec((1, 1), lambda b: (0, 0)),
        ],
        out_specs=pl.BlockSpec((H, W, S, Cc), lambda b: (0, 0, b, 0)),
        out_shape=jax.ShapeDtypeStruct((H, W, B, Cc), jnp.float32),
        scratch_shapes=[pltpu.VMEM((S * HW, Cc), jnp.float32)],
    )(xt, qmat.astype(_BF), wkv.astype(_BF), Wm1.astype(_BF), Wm2.astype(_BF),
      Wexp.astype(_BF), alpha2)
    return jnp.transpose(out, (2, 0, 1, 3))
